# Initial kernel scaffold; baseline (speedup 1.0000x reference)
import jax
import jax.numpy as jnp
from jax.experimental import pallas as pl


def kernel(node_feat, net_feat, pin_feat, pins_src, pins_dst, pinned_src, pinned_dst, W_conv, b_conv, W_lin, b_lin, b_nn):
    raise NotImplementedError("write your pallas kernel here")


# speedup vs baseline: 3.7218x; 3.7218x over previous
import functools

import jax
import jax.numpy as jnp
import numpy as np
from jax import lax
from jax.experimental import pallas as pl
from jax.experimental.pallas import tpu as pltpu
from jax.experimental.pallas import tpu_sc as plsc

N_NODES = 10000
N_NETS = 2000
E = 160000
D_NODE = 128
D_NET = 16
D_PIN = 16
D_OUT_NODE = 16
D_OUT_NET = 128

NC = 2
NS = 16
E_PER_CORE = E // NC
E_PER_TILE = E // (NC * NS)
CHUNK = 128
N_FULL = E_PER_TILE // CHUNK
TAIL = E_PER_TILE - N_FULL * CHUNK

_MESH = plsc.VectorSubcoreMesh(core_axis_name="c", subcore_axis_name="s")


def _sc1_body(pins_src, pins_dst, pinned_dst, pinned_src, net_feat, ones16, zc,
              cnt_src, cnt_dst, cnt_in, srcnet,
              idx_v, idx8_v, ones_v, rows_v, rows8_v, sem,
              cs_sh, cd_sh, ci_sh):
  c = lax.axis_index("c")
  s = lax.axis_index("s")
  base = c * E_PER_CORE + s * E_PER_TILE

  @pl.when(s < 10)
  def _():
    r0 = s * 1000
    pltpu.sync_copy(zc.at[pl.ds(r0, 1000)], cs_sh.at[pl.ds(r0, 1000)])
    pltpu.sync_copy(zc.at[pl.ds(r0, 1000)], ci_sh.at[pl.ds(r0, 1000)])
    r1 = s * 200
    pltpu.sync_copy(zc.at[pl.ds(r1, 200)], cd_sh.at[pl.ds(r1, 200)])

  pltpu.sync_copy(ones16, ones_v)
  plsc.subcore_barrier()

  def count_into(idx_hbm, sh):
    def step(j, carry):
      pltpu.sync_copy(idx_hbm.at[pl.ds(base + j * CHUNK, CHUNK)], idx_v)
      pltpu.sync_copy(ones_v, sh.at[idx_v], add=True)
      return carry
    lax.fori_loop(0, N_FULL, step, 0)
    pltpu.sync_copy(idx_hbm.at[pl.ds(base + N_FULL * CHUNK, TAIL)], idx8_v)
    pltpu.sync_copy(ones_v.at[pl.ds(0, TAIL)], sh.at[idx8_v], add=True)

  count_into(pins_src, cs_sh)
  count_into(pins_dst, cd_sh)
  count_into(pinned_dst, ci_sh)

  def gstep(j, carry):
    off = base + j * CHUNK
    pltpu.sync_copy(pinned_src.at[pl.ds(off, CHUNK)], idx_v)
    pltpu.async_copy(net_feat.at[idx_v], rows_v, sem).wait()
    pltpu.sync_copy(rows_v, srcnet.at[pl.ds(off, CHUNK)])
    return carry
  lax.fori_loop(0, N_FULL, gstep, 0)
  off = base + N_FULL * CHUNK
  pltpu.sync_copy(pinned_src.at[pl.ds(off, TAIL)], idx8_v)
  pltpu.async_copy(net_feat.at[idx8_v], rows8_v, sem).wait()
  pltpu.sync_copy(rows8_v, srcnet.at[pl.ds(off, TAIL)])

  plsc.subcore_barrier()

  @pl.when(s < 10)
  def _():
    r0 = s * 1000
    pltpu.sync_copy(cs_sh.at[pl.ds(r0, 1000)], cnt_src.at[c, pl.ds(r0, 1000)])
    pltpu.sync_copy(ci_sh.at[pl.ds(r0, 1000)], cnt_in.at[c, pl.ds(r0, 1000)])
    r1 = s * 200
    pltpu.sync_copy(cd_sh.at[pl.ds(r1, 200)], cnt_dst.at[c, pl.ds(r1, 200)])


_SC_PARAMS = pltpu.CompilerParams(use_tc_tiling_on_sc=False)

_sc1 = functools.partial(
    pl.kernel,
    mesh=_MESH,
    compiler_params=_SC_PARAMS,
    out_type=[
        jax.ShapeDtypeStruct((NC, N_NODES, 16), jnp.float32),
        jax.ShapeDtypeStruct((NC, N_NETS, 16), jnp.float32),
        jax.ShapeDtypeStruct((NC, N_NODES, 16), jnp.float32),
        jax.ShapeDtypeStruct((E, D_NET), jnp.float32),
    ],
    scratch_types=[
        pltpu.VMEM((CHUNK,), jnp.int32),
        pltpu.VMEM((TAIL,), jnp.int32),
        pltpu.VMEM((CHUNK, 16), jnp.float32),
        pltpu.VMEM((CHUNK, D_NET), jnp.float32),
        pltpu.VMEM((TAIL, D_NET), jnp.float32),
        pltpu.SemaphoreType.DMA,
        pltpu.VMEM_SHARED((N_NODES, 16), jnp.float32),
        pltpu.VMEM_SHARED((N_NETS, 16), jnp.float32),
        pltpu.VMEM_SHARED((N_NODES, 16), jnp.float32),
    ],
)(_sc1_body)


def _sc2_body(h, m, pins_src, pins_dst, pinned_dst, zbig, zsmall,
              agg, nacc,
              idxs_v, idxd_v, idx8a_v, idx8b_v,
              hrows_v, h8_v, mrows_v, m8_v, sem,
              agg_sh, nacc_sh):
  c = lax.axis_index("c")
  s = lax.axis_index("s")
  base = c * E_PER_CORE + s * E_PER_TILE

  @pl.when(s < 10)
  def _():
    r0 = s * 200
    pltpu.sync_copy(zbig.at[pl.ds(r0, 200)], agg_sh.at[pl.ds(r0, 200)])
    r1 = s * 1000
    pltpu.sync_copy(zsmall.at[pl.ds(r1, 1000)], nacc_sh.at[pl.ds(r1, 1000)])
  plsc.subcore_barrier()

  def step(j, carry):
    off = base + j * CHUNK
    pltpu.sync_copy(pins_src.at[pl.ds(off, CHUNK)], idxs_v)
    pltpu.sync_copy(pins_dst.at[pl.ds(off, CHUNK)], idxd_v)
    pltpu.async_copy(h.at[idxs_v], hrows_v, sem).wait()
    pltpu.sync_copy(hrows_v, agg_sh.at[idxd_v], add=True)
    pltpu.sync_copy(pinned_dst.at[pl.ds(off, CHUNK)], idxd_v)
    pltpu.sync_copy(m.at[pl.ds(off, CHUNK)], mrows_v)
    pltpu.sync_copy(mrows_v, nacc_sh.at[idxd_v], add=True)
    return carry
  lax.fori_loop(0, N_FULL, step, 0)

  off = base + N_FULL * CHUNK
  pltpu.sync_copy(pins_src.at[pl.ds(off, TAIL)], idx8a_v)
  pltpu.sync_copy(pins_dst.at[pl.ds(off, TAIL)], idx8b_v)
  pltpu.async_copy(h.at[idx8a_v], h8_v, sem).wait()
  pltpu.sync_copy(h8_v, agg_sh.at[idx8b_v], add=True)
  pltpu.sync_copy(pinned_dst.at[pl.ds(off, TAIL)], idx8b_v)
  pltpu.sync_copy(m.at[pl.ds(off, TAIL)], m8_v)
  pltpu.sync_copy(m8_v, nacc_sh.at[idx8b_v], add=True)

  plsc.subcore_barrier()

  @pl.when(s < 10)
  def _():
    r0 = s * 200
    pltpu.sync_copy(agg_sh.at[pl.ds(r0, 200)], agg.at[c, pl.ds(r0, 200)])
    r1 = s * 1000
    pltpu.sync_copy(nacc_sh.at[pl.ds(r1, 1000)], nacc.at[c, pl.ds(r1, 1000)])


_sc2 = functools.partial(
    pl.kernel,
    mesh=_MESH,
    compiler_params=_SC_PARAMS,
    out_type=[
        jax.ShapeDtypeStruct((NC, N_NETS, D_OUT_NET), jnp.float32),
        jax.ShapeDtypeStruct((NC, N_NODES, D_OUT_NODE), jnp.float32),
    ],
    scratch_types=[
        pltpu.VMEM((CHUNK,), jnp.int32),
        pltpu.VMEM((CHUNK,), jnp.int32),
        pltpu.VMEM((TAIL,), jnp.int32),
        pltpu.VMEM((TAIL,), jnp.int32),
        pltpu.VMEM((CHUNK, D_OUT_NET), jnp.float32),
        pltpu.VMEM((TAIL, D_OUT_NET), jnp.float32),
        pltpu.VMEM((CHUNK, D_OUT_NODE), jnp.float32),
        pltpu.VMEM((TAIL, D_OUT_NODE), jnp.float32),
        pltpu.SemaphoreType.DMA,
        pltpu.VMEM_SHARED((N_NETS, D_OUT_NET), jnp.float32),
        pltpu.VMEM_SHARED((N_NODES, D_OUT_NODE), jnp.float32),
    ],
)(_sc2_body)


_H_BLK = 1000


def _h_body(x_ref, c0_ref, c1_ref, w_ref, o_ref):
  cnt = c0_ref[...][:, :1] + c1_ref[...][:, :1]
  scale = lax.rsqrt(jnp.maximum(cnt, 1.0))
  o_ref[...] = jnp.dot(x_ref[...] * scale, w_ref[...],
                       preferred_element_type=jnp.float32)


def _h_call(node_feat, c0, c1, w):
  grid = N_NODES // _H_BLK
  return pl.pallas_call(
      _h_body,
      grid=(grid,),
      in_specs=[
          pl.BlockSpec((_H_BLK, D_NODE), lambda i: (i, 0)),
          pl.BlockSpec((_H_BLK, 16), lambda i: (i, 0)),
          pl.BlockSpec((_H_BLK, 16), lambda i: (i, 0)),
          pl.BlockSpec((D_NODE, D_OUT_NET), lambda i: (0, 0)),
      ],
      out_specs=pl.BlockSpec((_H_BLK, D_OUT_NET), lambda i: (i, 0)),
      out_shape=jax.ShapeDtypeStruct((N_NODES, D_OUT_NET), jnp.float32),
  )(node_feat, c0, c1, w)


_M_BLK = 1600


def _m_body(pin_ref, sn_ref, r_ref, s_ref, t2_ref, b_ref, o_ref):
  pin = pin_ref[...]
  sn = sn_ref[...]
  zr = jnp.dot(pin, r_ref[...], preferred_element_type=jnp.float32)
  zt = jnp.dot(sn, s_ref[...], preferred_element_type=jnp.float32)
  o_ref[...] = (jnp.dot(zr * zt, t2_ref[...], preferred_element_type=jnp.float32)
                + jnp.dot(sn, b_ref[...], preferred_element_type=jnp.float32))


def _m_call(pin_feat, srcnet, rmat, smat, t2, bmat):
  grid = E // _M_BLK
  return pl.pallas_call(
      _m_body,
      grid=(grid,),
      in_specs=[
          pl.BlockSpec((_M_BLK, D_PIN), lambda i: (i, 0)),
          pl.BlockSpec((_M_BLK, D_NET), lambda i: (i, 0)),
          pl.BlockSpec((D_PIN, D_PIN * D_NET), lambda i: (0, 0)),
          pl.BlockSpec((D_NET, D_PIN * D_NET), lambda i: (0, 0)),
          pl.BlockSpec((D_PIN * D_NET, D_OUT_NODE), lambda i: (0, 0)),
          pl.BlockSpec((D_NET, D_OUT_NODE), lambda i: (0, 0)),
      ],
      out_specs=pl.BlockSpec((_M_BLK, D_OUT_NODE), lambda i: (i, 0)),
      out_shape=jax.ShapeDtypeStruct((E, D_OUT_NODE), jnp.float32),
  )(pin_feat, srcnet, rmat, smat, t2, bmat)


def _net_body(a0_ref, a1_ref, c0_ref, c1_ref, b_ref, o_ref):
  agg = a0_ref[...] + a1_ref[...]
  deg = jnp.maximum(c0_ref[...][:, :1] + c1_ref[...][:, :1], 1.0)
  o_ref[...] = agg * lax.rsqrt(deg) + b_ref[...]


def _net_call(a0, a1, c0, c1, b):
  return pl.pallas_call(
      _net_body,
      out_shape=jax.ShapeDtypeStruct((N_NETS, D_OUT_NET), jnp.float32),
  )(a0, a1, c0, c1, b)


def _node_body(n0_ref, n1_ref, c0_ref, c1_ref, b_ref, o_ref):
  acc = n0_ref[...] + n1_ref[...]
  deg = jnp.maximum(c0_ref[...][:, :1] + c1_ref[...][:, :1], 1.0)
  o_ref[...] = acc / deg + b_ref[...]


def _node_call(n0, n1, c0, c1, b):
  return pl.pallas_call(
      _node_body,
      out_shape=jax.ShapeDtypeStruct((N_NODES, D_OUT_NODE), jnp.float32),
  )(n0, n1, c0, c1, b)


_R_NP = np.repeat(np.eye(D_PIN, dtype=np.float32), D_NET, axis=1)
_S_NP = np.tile(np.eye(D_NET, dtype=np.float32), (1, D_PIN))


@jax.jit
def kernel(node_feat, net_feat, pin_feat, pins_src, pins_dst, pinned_src,
           pinned_dst, W_conv, b_conv, W_lin, b_lin, b_nn):
  pins_src = pins_src.astype(jnp.int32)
  pins_dst = pins_dst.astype(jnp.int32)
  pinned_src = pinned_src.astype(jnp.int32)
  pinned_dst = pinned_dst.astype(jnp.int32)

  ones16 = jnp.ones((CHUNK, 16), jnp.float32)
  zc = jnp.zeros((N_NODES, 16), jnp.float32)

  cnt_src, cnt_dst, cnt_in, srcnet = _sc1(
      pins_src, pins_dst, pinned_dst, pinned_src, net_feat, ones16, zc)

  h = _h_call(node_feat, cnt_src[0], cnt_src[1], W_conv)

  t2 = W_lin.reshape(D_PIN * D_NET, D_OUT_NODE)
  bmat = b_lin.reshape(D_NET, D_OUT_NODE)
  m = _m_call(pin_feat, srcnet, jnp.asarray(_R_NP), jnp.asarray(_S_NP), t2,
              bmat)

  zbig = jnp.zeros((N_NETS, D_OUT_NET), jnp.float32)
  zsmall = jnp.zeros((N_NODES, D_OUT_NODE), jnp.float32)
  agg, nacc = _sc2(h, m, pins_src, pins_dst, pinned_dst, zbig, zsmall)

  net_out = _net_call(agg[0], agg[1], cnt_dst[0], cnt_dst[1],
                      b_conv.reshape(1, D_OUT_NET))
  node_out = _node_call(nacc[0], nacc[1], cnt_in[0], cnt_in[1],
                        b_nn.reshape(1, D_OUT_NODE))
  return (node_out, net_out)
